# baseline (speedup 1.0000x reference)
import functools

import jax
import jax.numpy as jnp
from jax import lax
from jax.experimental import pallas as pl
from jax.experimental.pallas import tpu as pltpu
from jax.experimental.pallas import tpu_sc as plsc

B = 16384
EMB = 32
NW = 32
RPW = B // NW
NCHUNK = RPW // 128
NSTREAMS = 8


def _shift(idx_v, offs_v):
    def body(c, carry):
        v = idx_v[c // 8, pl.ds((c % 8) * 16, 16)]
        offs_v[c // 4, pl.ds((c % 4) * 16, 16)] = (
            lax.shift_right_logical(v, 2))
        return carry
    lax.fori_loop(0, 32, body, 0)


def _gather_extract(tab_hbm, offs_v, idx_v, stage_v, drows_v, sem):
    def fire(s, buf):
        pltpu.make_async_copy(
            tab_hbm.at[offs_v.at[s]], stage_v.at[buf], sem).start()

    def wait(s, buf):
        pltpu.make_async_copy(
            tab_hbm.at[offs_v.at[s]], stage_v.at[buf], sem).wait()

    fire(0, 0)
    fire(1, 1)

    iota = lax.iota(jnp.int32, 16)

    for s in range(NSTREAMS):
        buf = s % 2
        wait(s, buf)

        for g in range(4):
            idx16 = idx_v[s // 2, pl.ds((s % 2) * 64 + g * 16, 16)]
            cbase = (idx16 & 3) * 32
            rows = iota + g * 16
            orows = rows + s * 64

            def extract(j, carry2, rows=rows, cbase=cbase, orows=orows,
                        buf=buf):
                vals = plsc.load_gather(stage_v.at[buf], [rows, cbase + j])
                plsc.store_scatter(
                    drows_v, [orows, jnp.zeros((16,), jnp.int32) + j], vals)
                return carry2

            lax.fori_loop(0, EMB, extract, 0)

        if s < NSTREAMS - 2:
            fire(s + 2, buf)


@functools.cache
def _build_sc_gather():
    mesh = plsc.VectorSubcoreMesh(core_axis_name="c", subcore_axis_name="s")

    @functools.partial(
        pl.kernel,
        mesh=mesh,
        compiler_params=pltpu.CompilerParams(needs_layout_passes=False),
        out_type=(
            jax.ShapeDtypeStruct((B, EMB), jnp.float32),
            jax.ShapeDtypeStruct((B, EMB), jnp.float32),
        ),
        scratch_types=[
            pltpu.VMEM((NCHUNK, 128), jnp.int32),
            pltpu.VMEM((NCHUNK, 128), jnp.int32),
            pltpu.VMEM((NSTREAMS, 64), jnp.int32),
            pltpu.VMEM((NSTREAMS, 64), jnp.int32),
            pltpu.VMEM((2, 64, 128), jnp.float32),
            pltpu.VMEM((RPW, EMB), jnp.float32),
            pltpu.SemaphoreType.DMA,
        ],
    )
    def sc_gather(uid_hbm, iid_hbm, utab_hbm, itab_hbm, uout_hbm, iout_hbm,
                  uidx_v, iidx_v, uoffs_v, ioffs_v, stage_v, drows_v, sem):
        wid = lax.axis_index("s") * 2 + lax.axis_index("c")
        base = wid * RPW
        pltpu.sync_copy(uid_hbm.at[pl.ds(wid * NCHUNK, NCHUNK)], uidx_v)
        pltpu.sync_copy(iid_hbm.at[pl.ds(wid * NCHUNK, NCHUNK)], iidx_v)
        _shift(uidx_v, uoffs_v)
        _shift(iidx_v, ioffs_v)
        _gather_extract(utab_hbm, uoffs_v, uidx_v, stage_v, drows_v, sem)
        pltpu.sync_copy(drows_v, uout_hbm.at[pl.ds(base, RPW)])
        _gather_extract(itab_hbm, ioffs_v, iidx_v, stage_v, drows_v, sem)
        pltpu.sync_copy(drows_v, iout_hbm.at[pl.ds(base, RPW)])

    return sc_gather


def _pack_body(x_ref, perm_ref, o_ref):
    zz = lax.dot_general(perm_ref[...], x_ref[...],
                         dimension_numbers=(((0,), (1,)), ((), ())),
                         preferred_element_type=jnp.float32)
    for k in range(4):
        o_ref[:, k * EMB:(k + 1) * EMB] = zz[k * 128:(k + 1) * 128, :]


def _pack(tab_t, n):
    nblk = -(-n // 512)
    r = jnp.arange(128)
    k = jnp.arange(4)
    rows = (4 * r[None, :] + k[:, None]).reshape(-1)
    perm = jnp.zeros((512, 512), jnp.float32).at[rows, jnp.arange(512)].set(1.0)
    return pl.pallas_call(
        _pack_body,
        grid=(nblk,),
        in_specs=[pl.BlockSpec((EMB, 512), lambda b: (0, b)),
                  pl.BlockSpec((512, 512), lambda b: (0, 0))],
        out_specs=pl.BlockSpec((128, 128), lambda b: (b, 0)),
        out_shape=jax.ShapeDtypeStruct((nblk * 128, 128), jnp.float32),
    )(tab_t, perm)


def _mlp_body(u_ref, v_ref, w0a, w0b, b0, w1, b1, w2, b2, w3, b3, wout, bout,
              o_ref):
    dot = functools.partial(jnp.dot, preferred_element_type=jnp.float32)
    x = jnp.maximum(dot(u_ref[...], w0a[...]) + dot(v_ref[...], w0b[...])
                    + b0[...], 0.0)
    x = jnp.maximum(dot(x, w1[...]) + b1[...], 0.0)
    x = jnp.maximum(dot(x, w2[...]) + b2[...], 0.0)
    x = jnp.maximum(dot(x, w3[...]) + b3[...], 0.0)
    o_ref[...] = jax.nn.sigmoid(dot(x, wout[...]) + bout[...])


def _mlp(u, v, w0a, w0b, b0, w1, b1, w2, b2, w3, b3, wout, bout):
    blk = 2048
    grid = (B // blk,)

    def full(shape):
        return pl.BlockSpec(shape, lambda i: (0, 0))

    return pl.pallas_call(
        _mlp_body,
        grid=grid,
        in_specs=[
            pl.BlockSpec((blk, EMB), lambda i: (i, 0)),
            pl.BlockSpec((blk, EMB), lambda i: (i, 0)),
            full((EMB, 64)), full((EMB, 64)), full((1, 64)),
            full((64, 32)), full((1, 32)),
            full((32, 16)), full((1, 16)),
            full((16, 8)), full((1, 8)),
            full((8, 1)), full((1, 1)),
        ],
        out_specs=pl.BlockSpec((blk, 1), lambda i: (i, 0)),
        out_shape=jax.ShapeDtypeStruct((B, 1), jnp.float32),
    )(u, v, w0a, w0b, b0, w1, b1, w2, b2, w3, b3, wout, bout)


def kernel(user_id, item_id, user_table, item_table, W0, b0, W1, b1, W2, b2,
           W3, b3, Wout, bout):
    uid2 = user_id.astype(jnp.int32).reshape(B // 128, 128)
    iid2 = item_id.astype(jnp.int32).reshape(B // 128, 128)
    ut4 = _pack(user_table.T, user_table.shape[0])
    it4 = _pack(item_table.T, item_table.shape[0])
    uemb, iemb = _build_sc_gather()(uid2, iid2, ut4, it4)
    return _mlp(uemb, iemb,
                W0[:EMB], W0[EMB:], b0.reshape(1, -1),
                W1, b1.reshape(1, -1), W2, b2.reshape(1, -1),
                W3, b3.reshape(1, -1), Wout, bout.reshape(1, -1))


# speedup vs baseline: 4.2681x; 4.2681x over previous
import functools

import jax
import jax.numpy as jnp
from jax import lax
from jax.experimental import pallas as pl
from jax.experimental.pallas import tpu as pltpu
from jax.experimental.pallas import tpu_sc as plsc

B = 16384
EMB = 32
NW = 32
RPW = B // NW


@functools.cache
def _build_sc_gather():
    mesh = plsc.VectorSubcoreMesh(core_axis_name="c", subcore_axis_name="s")

    @functools.partial(
        pl.kernel,
        mesh=mesh,
        compiler_params=pltpu.CompilerParams(needs_layout_passes=False),
        out_type=(
            jax.ShapeDtypeStruct((B, EMB), jnp.float32),
            jax.ShapeDtypeStruct((B, EMB), jnp.float32),
        ),
        scratch_types=[
            pltpu.VMEM((RPW + 16,), jnp.int32),
            pltpu.VMEM((RPW + 16,), jnp.int32),
            pltpu.VMEM((RPW // 2, EMB), jnp.float32),
            pltpu.VMEM((RPW // 2, EMB), jnp.float32),
            pltpu.SemaphoreType.DMA,
        ],
    )
    def sc_gather(uid_hbm, iid_hbm, utab_hbm, itab_hbm, uout_hbm, iout_hbm,
                  uidx_v, iidx_v, urows_v, irows_v, sem):
        ch = RPW // 2
        wid = lax.axis_index("s") * 2 + lax.axis_index("c")
        base = wid * RPW
        pltpu.sync_copy(uid_hbm.at[pl.ds(base, RPW)],
                        uidx_v.at[pl.ds(0, RPW)])
        pltpu.sync_copy(iid_hbm.at[pl.ds(base, RPW)],
                        iidx_v.at[pl.ds(0, RPW)])

        for r in range(2):
            def body(i, carry, r=r):
                j = i - r * ch
                uix = uidx_v[pl.ds(i, 16)][0]
                iix = iidx_v[pl.ds(i, 16)][0]
                pltpu.make_async_copy(
                    utab_hbm.at[pl.ds(uix, 1)],
                    urows_v.at[pl.ds(j, 1)], sem).start()
                pltpu.make_async_copy(
                    itab_hbm.at[pl.ds(iix, 1)],
                    irows_v.at[pl.ds(j, 1)], sem).start()
                return carry

            lax.fori_loop(r * ch, (r + 1) * ch, body, 0)
            pltpu.make_async_copy(
                utab_hbm.at[pl.ds(0, ch)], urows_v, sem).wait()
            pltpu.make_async_copy(
                itab_hbm.at[pl.ds(0, ch)], irows_v, sem).wait()
            pltpu.sync_copy(urows_v, uout_hbm.at[pl.ds(base + r * ch, ch)])
            pltpu.sync_copy(irows_v, iout_hbm.at[pl.ds(base + r * ch, ch)])

    return sc_gather


def _mlp_body(u_ref, v_ref, w0a, w0b, b0, w1, b1, w2, b2, w3, b3, wout, bout,
              o_ref):
    dot = functools.partial(jnp.dot, preferred_element_type=jnp.float32)
    x = jnp.maximum(dot(u_ref[...], w0a[...]) + dot(v_ref[...], w0b[...])
                    + b0[...], 0.0)
    x = jnp.maximum(dot(x, w1[...]) + b1[...], 0.0)
    x = jnp.maximum(dot(x, w2[...]) + b2[...], 0.0)
    x = jnp.maximum(dot(x, w3[...]) + b3[...], 0.0)
    o_ref[...] = jax.nn.sigmoid(dot(x, wout[...]) + bout[...])


def _mlp(u, v, w0a, w0b, b0, w1, b1, w2, b2, w3, b3, wout, bout):
    blk = 2048
    grid = (B // blk,)

    def full(shape):
        return pl.BlockSpec(shape, lambda i: (0, 0))

    return pl.pallas_call(
        _mlp_body,
        grid=grid,
        in_specs=[
            pl.BlockSpec((blk, EMB), lambda i: (i, 0)),
            pl.BlockSpec((blk, EMB), lambda i: (i, 0)),
            full((EMB, 64)), full((EMB, 64)), full((1, 64)),
            full((64, 32)), full((1, 32)),
            full((32, 16)), full((1, 16)),
            full((16, 8)), full((1, 8)),
            full((8, 1)), full((1, 1)),
        ],
        out_specs=pl.BlockSpec((blk, 1), lambda i: (i, 0)),
        out_shape=jax.ShapeDtypeStruct((B, 1), jnp.float32),
    )(u, v, w0a, w0b, b0, w1, b1, w2, b2, w3, b3, wout, bout)


def kernel(user_id, item_id, user_table, item_table, W0, b0, W1, b1, W2, b2,
           W3, b3, Wout, bout):
    uemb, iemb = _build_sc_gather()(
        user_id.astype(jnp.int32), item_id.astype(jnp.int32),
        user_table, item_table)
    return _mlp(uemb, iemb,
                W0[:EMB], W0[EMB:], b0.reshape(1, -1),
                W1, b1.reshape(1, -1), W2, b2.reshape(1, -1),
                W3, b3.reshape(1, -1), Wout, bout.reshape(1, -1))
